# Initial kernel scaffold; baseline (speedup 1.0000x reference)
import jax
import jax.numpy as jnp
from jax.experimental import pallas as pl


def kernel(node_emb, edge_index, edge_type, comp0, basis0, root0, bias0, g0, b0, comp1, basis1, root1, bias1, g1, b1):
    raise NotImplementedError("write your pallas kernel here")


# speedup vs baseline: 2.2308x; 2.2308x over previous
import functools

import jax
import jax.numpy as jnp
from jax import lax
from jax.experimental import pallas as pl
from jax.experimental.pallas import tpu as pltpu
from jax.experimental.pallas import tpu_sc as plsc

N = 10000
R = 24
D = 128

NC = 2
NS = 16
NW = NC * NS
B = 128

NR = N * R
NRP = 262144
CNT_SLICE = NRP // NS
AGG_ROWS = 10112
AGG_SLICE = AGG_ROWS // NS


def _sc_mesh():
    return plsc.VectorSubcoreMesh(core_axis_name="c", subcore_axis_name="s")


def _counts_sc(e_pad):
    chunk = e_pad // NW
    nbatch = chunk // B

    @functools.partial(
        pl.kernel,
        out_type=(
            jax.ShapeDtypeStruct((NRP,), jnp.float32),
            jax.ShapeDtypeStruct((NRP,), jnp.float32),
        ),
        mesh=_sc_mesh(),
        scratch_types=[
            pltpu.VMEM_SHARED((NRP,), jnp.float32),
            pltpu.VMEM((B,), jnp.int32),
            pltpu.VMEM((B,), jnp.int32),
            pltpu.VMEM((1, B), jnp.int32),
            pltpu.VMEM((B,), jnp.float32),
        ],
    )
    def k(dst_hbm, typ_hbm, zeros_hbm, cnt0_hbm, cnt1_hbm,
          cnt_sh, dbuf, tbuf, cidbuf, ones):
        c = lax.axis_index("c")
        s = lax.axis_index("s")
        wid = s * NC + c
        base0 = wid * chunk

        pltpu.sync_copy(zeros_hbm.at[pl.ds(s * CNT_SLICE, CNT_SLICE)],
                        cnt_sh.at[pl.ds(s * CNT_SLICE, CNT_SLICE)])
        for g in range(B // 16):
            ones[pl.ds(16 * g, 16)] = jnp.ones((16,), jnp.float32)
        plsc.subcore_barrier()

        def body(b, _):
            base = base0 + b * B
            pltpu.sync_copy(dst_hbm.at[pl.ds(base, B)], dbuf)
            pltpu.sync_copy(typ_hbm.at[pl.ds(base, B)], tbuf)
            for g in range(B // 16):
                dv = dbuf[pl.ds(16 * g, 16)]
                tv = tbuf[pl.ds(16 * g, 16)]
                cidbuf[0, pl.ds(16 * g, 16)] = dv * R + tv
            pltpu.sync_copy(ones, cnt_sh.at[cidbuf.at[0]], add=True)
            return ()

        lax.fori_loop(0, nbatch, body, ())
        plsc.subcore_barrier()

        sl = pl.ds(s * CNT_SLICE, CNT_SLICE)

        @pl.when(c == 0)
        def _():
            pltpu.sync_copy(cnt_sh.at[sl], cnt0_hbm.at[sl])

        @pl.when(c == 1)
        def _():
            pltpu.sync_copy(cnt_sh.at[sl], cnt1_hbm.at[sl])

    return k


def _aggregate_sc(e_pad):
    chunk = e_pad // NW
    nbatch = chunk // B

    @functools.partial(
        pl.kernel,
        out_type=(
            jax.ShapeDtypeStruct((AGG_ROWS, D), jnp.float32),
            jax.ShapeDtypeStruct((AGG_ROWS, D), jnp.float32),
        ),
        mesh=_sc_mesh(),
        scratch_types=[
            pltpu.VMEM_SHARED((AGG_ROWS, D), jnp.float32),
            pltpu.VMEM((B,), jnp.int32),
            pltpu.VMEM((1, B), jnp.int32),
            pltpu.VMEM((B,), jnp.int32),
            pltpu.VMEM((1, B), jnp.int32),
            pltpu.VMEM((1, B), jnp.int32),
            pltpu.VMEM((B, D), jnp.float32),
            pltpu.VMEM((B,), jnp.float32),
            pltpu.VMEM((B,), jnp.float32),
            pltpu.VMEM((B,), jnp.float32),
            pltpu.SemaphoreType.DMA,
        ],
    )
    def k(src_hbm, dst_hbm, typ_hbm, h_hbm, cnt0_hbm, cnt1_hbm, zeros_hbm,
          agg0_hbm, agg1_hbm,
          agg_sh, sbuf, dbuf, tbuf, gidx, cidx, rows, c0b, c1b, wbuf, sem):
        c = lax.axis_index("c")
        s = lax.axis_index("s")
        wid = s * NC + c
        base0 = wid * chunk

        rsl = pl.ds(s * AGG_SLICE, AGG_SLICE)
        pltpu.sync_copy(zeros_hbm.at[rsl], agg_sh.at[rsl])
        plsc.subcore_barrier()

        def body(b, _):
            base = base0 + b * B
            pltpu.sync_copy(src_hbm.at[pl.ds(base, B)], sbuf)
            pltpu.sync_copy(dst_hbm.at[pl.ds(base, B)], dbuf.at[0])
            pltpu.sync_copy(typ_hbm.at[pl.ds(base, B)], tbuf)
            for g in range(B // 16):
                sl16 = pl.ds(16 * g, 16)
                sv = sbuf[sl16]
                dv = dbuf[0, sl16]
                tv = tbuf[sl16]
                gidx[0, sl16] = tv * N + sv
                cidx[0, sl16] = dv * R + tv
            cp1 = pltpu.async_copy(h_hbm.at[gidx.at[0]], rows, sem)
            cp2 = pltpu.async_copy(cnt0_hbm.at[cidx.at[0]], c0b, sem)
            cp3 = pltpu.async_copy(cnt1_hbm.at[cidx.at[0]], c1b, sem)
            cp1.wait()
            cp2.wait()
            cp3.wait()
            for g in range(B // 16):
                sl16 = pl.ds(16 * g, 16)
                wbuf[sl16] = 1.0 / (c0b[sl16] + c1b[sl16])

            for g in range(B // 16):
                wv = wbuf[pl.ds(16 * g, 16)]
                for l in range(16):
                    w = wv[l]
                    e_row = 16 * g + l
                    for j in range(D // 16):
                        cs = pl.ds(16 * j, 16)
                        rows[e_row, cs] = rows[e_row, cs] * w
            pltpu.sync_copy(rows, agg_sh.at[dbuf.at[0]], add=True)
            return ()

        lax.fori_loop(0, nbatch, body, ())
        plsc.subcore_barrier()

        @pl.when(c == 0)
        def _():
            pltpu.sync_copy(agg_sh.at[rsl], agg0_hbm.at[rsl])

        @pl.when(c == 1)
        def _():
            pltpu.sync_copy(agg_sh.at[rsl], agg1_hbm.at[rsl])

    return k


def _w_combine(comp, basis2d):
    def body(c_ref, b_ref, o_ref):
        o_ref[...] = jnp.dot(c_ref[...], b_ref[...],
                             preferred_element_type=jnp.float32)

    return pl.pallas_call(
        body,
        out_shape=jax.ShapeDtypeStruct((R, D * D), jnp.float32),
    )(comp, basis2d)


_BN_H = 2000


def _h_all(x, w3d):
    def body(x_ref, w_ref, o_ref):
        o_ref[0] = jnp.dot(x_ref[...], w_ref[0],
                           preferred_element_type=jnp.float32)

    return pl.pallas_call(
        body,
        grid=(N // _BN_H, R),
        in_specs=[
            pl.BlockSpec((_BN_H, D), lambda n, r: (n, 0)),
            pl.BlockSpec((1, D, D), lambda n, r: (r, 0, 0)),
        ],
        out_specs=pl.BlockSpec((1, _BN_H, D), lambda n, r: (r, n, 0)),
        out_shape=jax.ShapeDtypeStruct((R, N, D), jnp.float32),
    )(x, w3d)


_BN_F = 2000


def _finish(x, agg0, agg1, root, bias, g, b):
    def body(x_ref, a0_ref, a1_ref, r_ref, bias_ref, g_ref, b_ref, o_ref):
        y = (a0_ref[...] + a1_ref[...]
             + jnp.dot(x_ref[...], r_ref[...],
                       preferred_element_type=jnp.float32)
             + bias_ref[0])
        m = jnp.mean(y, axis=-1, keepdims=True)
        yc = y - m
        v = jnp.mean(yc * yc, axis=-1, keepdims=True)
        o = yc / jnp.sqrt(v + 1e-5) * g_ref[0] + b_ref[0]
        o_ref[...] = jnp.maximum(o, 0.0)

    return pl.pallas_call(
        body,
        grid=(N // _BN_F,),
        in_specs=[
            pl.BlockSpec((_BN_F, D), lambda n: (n, 0)),
            pl.BlockSpec((_BN_F, D), lambda n: (n, 0)),
            pl.BlockSpec((_BN_F, D), lambda n: (n, 0)),
            pl.BlockSpec((D, D), lambda n: (0, 0)),
            pl.BlockSpec((1, D), lambda n: (0, 0)),
            pl.BlockSpec((1, D), lambda n: (0, 0)),
            pl.BlockSpec((1, D), lambda n: (0, 0)),
        ],
        out_specs=pl.BlockSpec((_BN_F, D), lambda n: (n, 0)),
        out_shape=jax.ShapeDtypeStruct((N, D), jnp.float32),
    )(x, agg0[:N], agg1[:N], root, bias.reshape(1, D), g.reshape(1, D),
      b.reshape(1, D))


def kernel(node_emb, edge_index, edge_type,
           comp0, basis0, root0, bias0, g0, b0,
           comp1, basis1, root1, bias1, g1, b1):
    e = edge_index.shape[1]
    e_pad = ((e + NW * B - 1) // (NW * B)) * (NW * B)
    pad = e_pad - e

    src = jnp.pad(edge_index[0].astype(jnp.int32), (0, pad))
    dst = jnp.pad(edge_index[1].astype(jnp.int32), (0, pad),
                  constant_values=N)
    typ = jnp.pad(edge_type.astype(jnp.int32), (0, pad))

    zeros_cnt = jnp.zeros((NRP,), jnp.float32)
    zeros_agg = jnp.zeros((AGG_ROWS, D), jnp.float32)

    cnt0, cnt1 = _counts_sc(e_pad)(dst, typ, zeros_cnt)

    agg_fn = _aggregate_sc(e_pad)

    def layer(x, comp, basis, root, bias, g, b):
        w3d = _w_combine(comp, basis.reshape(R, D * D)).reshape(R, D, D)
        h = _h_all(x, w3d).reshape(R * N, D)
        a0, a1 = agg_fn(src, dst, typ, h, cnt0, cnt1, zeros_agg)
        return _finish(x, a0, a1, root, bias, g, b)

    x1 = layer(node_emb, comp0, basis0, root0, bias0, g0, b0)
    return layer(x1, comp1, basis1, root1, bias1, g1, b1)


# speedup vs baseline: 2.2726x; 1.0187x over previous
import functools

import jax
import jax.numpy as jnp
from jax import lax
from jax.experimental import pallas as pl
from jax.experimental.pallas import tpu as pltpu
from jax.experimental.pallas import tpu_sc as plsc

N = 10000
R = 24
D = 128

NC = 2
NS = 16
NW = NC * NS
B = 128

NR = N * R
NRP = 262144
CNT_SLICE = NRP // NS
AGG_ROWS = 10112
AGG_SLICE = AGG_ROWS // NS


def _sc_mesh():
    return plsc.VectorSubcoreMesh(core_axis_name="c", subcore_axis_name="s")


def _counts_sc(e_pad):
    chunk = e_pad // NW
    nbatch = chunk // B

    @functools.partial(
        pl.kernel,
        out_type=(
            jax.ShapeDtypeStruct((NRP,), jnp.float32),
            jax.ShapeDtypeStruct((NRP,), jnp.float32),
            jax.ShapeDtypeStruct((e_pad,), jnp.int32),
            jax.ShapeDtypeStruct((e_pad,), jnp.int32),
        ),
        mesh=_sc_mesh(),
        scratch_types=[
            pltpu.VMEM_SHARED((NRP,), jnp.float32),
            pltpu.VMEM((B,), jnp.int32),
            pltpu.VMEM((B,), jnp.int32),
            pltpu.VMEM((B,), jnp.int32),
            pltpu.VMEM((1, B), jnp.int32),
            pltpu.VMEM((1, B), jnp.int32),
            pltpu.VMEM((B,), jnp.float32),
        ],
    )
    def k(src_hbm, dst_hbm, typ_hbm, zeros_hbm,
          cnt0_hbm, cnt1_hbm, gid_hbm, cid_hbm,
          cnt_sh, sbuf, dbuf, tbuf, gidbuf, cidbuf, ones):
        c = lax.axis_index("c")
        s = lax.axis_index("s")
        wid = s * NC + c
        base0 = wid * chunk

        pltpu.sync_copy(zeros_hbm.at[pl.ds(s * CNT_SLICE, CNT_SLICE)],
                        cnt_sh.at[pl.ds(s * CNT_SLICE, CNT_SLICE)])
        for g in range(B // 16):
            ones[pl.ds(16 * g, 16)] = jnp.ones((16,), jnp.float32)
        plsc.subcore_barrier()

        def body(b, _):
            base = base0 + b * B
            pltpu.sync_copy(src_hbm.at[pl.ds(base, B)], sbuf)
            pltpu.sync_copy(dst_hbm.at[pl.ds(base, B)], dbuf)
            pltpu.sync_copy(typ_hbm.at[pl.ds(base, B)], tbuf)
            for g in range(B // 16):
                sl16 = pl.ds(16 * g, 16)
                sv = sbuf[sl16]
                dv = dbuf[sl16]
                tv = tbuf[sl16]
                gidbuf[0, sl16] = tv * N + sv
                cidbuf[0, sl16] = dv * R + tv
            pltpu.sync_copy(ones, cnt_sh.at[cidbuf.at[0]], add=True)
            pltpu.sync_copy(gidbuf.at[0], gid_hbm.at[pl.ds(base, B)])
            pltpu.sync_copy(cidbuf.at[0], cid_hbm.at[pl.ds(base, B)])
            return ()

        lax.fori_loop(0, nbatch, body, ())
        plsc.subcore_barrier()

        sl = pl.ds(s * CNT_SLICE, CNT_SLICE)

        @pl.when(c == 0)
        def _():
            pltpu.sync_copy(cnt_sh.at[sl], cnt0_hbm.at[sl])

        @pl.when(c == 1)
        def _():
            pltpu.sync_copy(cnt_sh.at[sl], cnt1_hbm.at[sl])

    return k


def _weights_sc(e_pad):
    chunk = e_pad // NW
    nbatch = chunk // B

    @functools.partial(
        pl.kernel,
        out_type=jax.ShapeDtypeStruct((e_pad,), jnp.float32),
        mesh=_sc_mesh(),
        scratch_types=[
            pltpu.VMEM_SHARED((NRP,), jnp.float32),
            pltpu.VMEM((CNT_SLICE,), jnp.float32),
            pltpu.VMEM((CNT_SLICE,), jnp.float32),
            pltpu.VMEM((CNT_SLICE,), jnp.float32),
            pltpu.VMEM((1, B), jnp.int32),
            pltpu.VMEM((B,), jnp.float32),
        ],
    )
    def k(cnt0_hbm, cnt1_hbm, cid_hbm, w_hbm,
          wbkt_sh, c0s, c1s, ws, cidbuf, wv):
        c = lax.axis_index("c")
        s = lax.axis_index("s")
        wid = s * NC + c
        base0 = wid * chunk

        sl = pl.ds(s * CNT_SLICE, CNT_SLICE)
        pltpu.sync_copy(cnt0_hbm.at[sl], c0s)
        pltpu.sync_copy(cnt1_hbm.at[sl], c1s)
        for i in range(CNT_SLICE // 16):
            s16 = pl.ds(16 * i, 16)
            ws[s16] = 1.0 / (c0s[s16] + c1s[s16])
        pltpu.sync_copy(ws, wbkt_sh.at[sl])
        plsc.subcore_barrier()

        def body(b, _):
            base = base0 + b * B
            pltpu.sync_copy(cid_hbm.at[pl.ds(base, B)], cidbuf.at[0])
            pltpu.sync_copy(wbkt_sh.at[cidbuf.at[0]], wv)
            pltpu.sync_copy(wv, w_hbm.at[pl.ds(base, B)])
            return ()

        lax.fori_loop(0, nbatch, body, ())

    return k


def _aggregate_sc(e_pad):
    chunk = e_pad // NW
    nbatch = chunk // B

    @functools.partial(
        pl.kernel,
        out_type=(
            jax.ShapeDtypeStruct((AGG_ROWS, D), jnp.float32),
            jax.ShapeDtypeStruct((AGG_ROWS, D), jnp.float32),
        ),
        mesh=_sc_mesh(),
        scratch_types=[
            pltpu.VMEM_SHARED((AGG_ROWS, D), jnp.float32),
            pltpu.VMEM((1, B), jnp.int32),
            pltpu.VMEM((1, B), jnp.int32),
            pltpu.VMEM((B,), jnp.float32),
            pltpu.VMEM((B, D), jnp.float32),
            pltpu.SemaphoreType.DMA,
        ],
    )
    def k(gid_hbm, dst_hbm, w_hbm, h_hbm, zeros_hbm,
          agg0_hbm, agg1_hbm,
          agg_sh, gidx, dbuf, wbuf, rows, sem):
        c = lax.axis_index("c")
        s = lax.axis_index("s")
        wid = s * NC + c
        base0 = wid * chunk

        rsl = pl.ds(s * AGG_SLICE, AGG_SLICE)
        pltpu.sync_copy(zeros_hbm.at[rsl], agg_sh.at[rsl])
        plsc.subcore_barrier()

        def body(b, _):
            base = base0 + b * B
            pltpu.sync_copy(gid_hbm.at[pl.ds(base, B)], gidx.at[0])
            pltpu.sync_copy(dst_hbm.at[pl.ds(base, B)], dbuf.at[0])
            pltpu.sync_copy(w_hbm.at[pl.ds(base, B)], wbuf)
            cp = pltpu.async_copy(h_hbm.at[gidx.at[0]], rows, sem)
            cp.wait()
            for g in range(B // 16):
                wv = wbuf[pl.ds(16 * g, 16)]
                for l in range(16):
                    w = wv[l]
                    e_row = 16 * g + l
                    for j in range(D // 16):
                        cs = pl.ds(16 * j, 16)
                        rows[e_row, cs] = rows[e_row, cs] * w
            pltpu.sync_copy(rows, agg_sh.at[dbuf.at[0]], add=True)
            return ()

        lax.fori_loop(0, nbatch, body, ())
        plsc.subcore_barrier()

        @pl.when(c == 0)
        def _():
            pltpu.sync_copy(agg_sh.at[rsl], agg0_hbm.at[rsl])

        @pl.when(c == 1)
        def _():
            pltpu.sync_copy(agg_sh.at[rsl], agg1_hbm.at[rsl])

    return k


def _w_combine(comp, basis2d):
    def body(c_ref, b_ref, o_ref):
        o_ref[...] = jnp.dot(c_ref[...], b_ref[...],
                             preferred_element_type=jnp.float32)

    return pl.pallas_call(
        body,
        out_shape=jax.ShapeDtypeStruct((R, D * D), jnp.float32),
    )(comp, basis2d)


_BN_H = 2000


def _h_all(x, w3d):
    def body(x_ref, w_ref, o_ref):
        o_ref[0] = jnp.dot(x_ref[...], w_ref[0],
                           preferred_element_type=jnp.float32)

    return pl.pallas_call(
        body,
        grid=(N // _BN_H, R),
        in_specs=[
            pl.BlockSpec((_BN_H, D), lambda n, r: (n, 0)),
            pl.BlockSpec((1, D, D), lambda n, r: (r, 0, 0)),
        ],
        out_specs=pl.BlockSpec((1, _BN_H, D), lambda n, r: (r, n, 0)),
        out_shape=jax.ShapeDtypeStruct((R, N, D), jnp.float32),
    )(x, w3d)


_BN_F = 2000


def _finish(x, agg0, agg1, root, bias, g, b):
    def body(x_ref, a0_ref, a1_ref, r_ref, bias_ref, g_ref, b_ref, o_ref):
        y = (a0_ref[...] + a1_ref[...]
             + jnp.dot(x_ref[...], r_ref[...],
                       preferred_element_type=jnp.float32)
             + bias_ref[0])
        m = jnp.mean(y, axis=-1, keepdims=True)
        yc = y - m
        v = jnp.mean(yc * yc, axis=-1, keepdims=True)
        o = yc / jnp.sqrt(v + 1e-5) * g_ref[0] + b_ref[0]
        o_ref[...] = jnp.maximum(o, 0.0)

    return pl.pallas_call(
        body,
        grid=(N // _BN_F,),
        in_specs=[
            pl.BlockSpec((_BN_F, D), lambda n: (n, 0)),
            pl.BlockSpec((_BN_F, D), lambda n: (n, 0)),
            pl.BlockSpec((_BN_F, D), lambda n: (n, 0)),
            pl.BlockSpec((D, D), lambda n: (0, 0)),
            pl.BlockSpec((1, D), lambda n: (0, 0)),
            pl.BlockSpec((1, D), lambda n: (0, 0)),
            pl.BlockSpec((1, D), lambda n: (0, 0)),
        ],
        out_specs=pl.BlockSpec((_BN_F, D), lambda n: (n, 0)),
        out_shape=jax.ShapeDtypeStruct((N, D), jnp.float32),
    )(x, agg0[:N], agg1[:N], root, bias.reshape(1, D), g.reshape(1, D),
      b.reshape(1, D))


def kernel(node_emb, edge_index, edge_type,
           comp0, basis0, root0, bias0, g0, b0,
           comp1, basis1, root1, bias1, g1, b1):
    e = edge_index.shape[1]
    e_pad = ((e + NW * B - 1) // (NW * B)) * (NW * B)
    pad = e_pad - e

    src = jnp.pad(edge_index[0].astype(jnp.int32), (0, pad))
    dst = jnp.pad(edge_index[1].astype(jnp.int32), (0, pad),
                  constant_values=N)
    typ = jnp.pad(edge_type.astype(jnp.int32), (0, pad))

    zeros_cnt = jnp.zeros((NRP,), jnp.float32)
    zeros_agg = jnp.zeros((AGG_ROWS, D), jnp.float32)

    cnt0, cnt1, gid, cid = _counts_sc(e_pad)(src, dst, typ, zeros_cnt)
    w = _weights_sc(e_pad)(cnt0, cnt1, cid)

    agg_fn = _aggregate_sc(e_pad)

    def layer(x, comp, basis, root, bias, g, b):
        w3d = _w_combine(comp, basis.reshape(R, D * D)).reshape(R, D, D)
        h = _h_all(x, w3d).reshape(R * N, D)
        a0, a1 = agg_fn(gid, dst, w, h, zeros_agg)
        return _finish(x, a0, a1, root, bias, g, b)

    x1 = layer(node_emb, comp0, basis0, root0, bias0, g0, b0)
    return layer(x1, comp1, basis1, root1, bias1, g1, b1)


# speedup vs baseline: 2.9109x; 1.2809x over previous
import functools

import jax
import jax.numpy as jnp
from jax import lax
from jax.experimental import pallas as pl
from jax.experimental.pallas import tpu as pltpu
from jax.experimental.pallas import tpu_sc as plsc

N = 10000
R = 24
D = 128

NC = 2
NS = 16
NW = NC * NS
B = 128

NR = N * R
NRP = 262144
CNT_SLICE = NRP // NS
AGG_ROWS = 10112
AGG_SLICE = AGG_ROWS // NS


def _sc_mesh():
    return plsc.VectorSubcoreMesh(core_axis_name="c", subcore_axis_name="s")


def _counts_sc(e_pad):
    chunk = e_pad // NW
    nbatch = chunk // B

    @functools.partial(
        pl.kernel,
        out_type=(
            jax.ShapeDtypeStruct((NRP,), jnp.float32),
            jax.ShapeDtypeStruct((NRP,), jnp.float32),
            jax.ShapeDtypeStruct((e_pad,), jnp.int32),
            jax.ShapeDtypeStruct((e_pad,), jnp.int32),
        ),
        mesh=_sc_mesh(),
        scratch_types=[
            pltpu.VMEM_SHARED((NRP,), jnp.float32),
            pltpu.VMEM((B,), jnp.int32),
            pltpu.VMEM((B,), jnp.int32),
            pltpu.VMEM((B,), jnp.int32),
            pltpu.VMEM((1, B), jnp.int32),
            pltpu.VMEM((1, B), jnp.int32),
            pltpu.VMEM((B,), jnp.float32),
        ],
    )
    def k(src_hbm, dst_hbm, typ_hbm, zeros_hbm,
          cnt0_hbm, cnt1_hbm, gid_hbm, cid_hbm,
          cnt_sh, sbuf, dbuf, tbuf, gidbuf, cidbuf, ones):
        c = lax.axis_index("c")
        s = lax.axis_index("s")
        wid = s * NC + c
        base0 = wid * chunk

        pltpu.sync_copy(zeros_hbm.at[pl.ds(s * CNT_SLICE, CNT_SLICE)],
                        cnt_sh.at[pl.ds(s * CNT_SLICE, CNT_SLICE)])
        for g in range(B // 16):
            ones[pl.ds(16 * g, 16)] = jnp.ones((16,), jnp.float32)
        plsc.subcore_barrier()

        def body(b, _):
            base = base0 + b * B
            pltpu.sync_copy(src_hbm.at[pl.ds(base, B)], sbuf)
            pltpu.sync_copy(dst_hbm.at[pl.ds(base, B)], dbuf)
            pltpu.sync_copy(typ_hbm.at[pl.ds(base, B)], tbuf)
            for g in range(B // 16):
                sl16 = pl.ds(16 * g, 16)
                sv = sbuf[sl16]
                dv = dbuf[sl16]
                tv = tbuf[sl16]
                gidbuf[0, sl16] = tv * N + sv
                cidbuf[0, sl16] = dv * R + tv
            pltpu.sync_copy(ones, cnt_sh.at[cidbuf.at[0]], add=True)
            pltpu.sync_copy(gidbuf.at[0], gid_hbm.at[pl.ds(base, B)])
            pltpu.sync_copy(cidbuf.at[0], cid_hbm.at[pl.ds(base, B)])
            return ()

        lax.fori_loop(0, nbatch, body, ())
        plsc.subcore_barrier()

        sl = pl.ds(s * CNT_SLICE, CNT_SLICE)

        @pl.when(c == 0)
        def _():
            pltpu.sync_copy(cnt_sh.at[sl], cnt0_hbm.at[sl])

        @pl.when(c == 1)
        def _():
            pltpu.sync_copy(cnt_sh.at[sl], cnt1_hbm.at[sl])

    return k


def _weights_sc(e_pad):
    chunk = e_pad // NW
    nbatch = chunk // B

    @functools.partial(
        pl.kernel,
        out_type=jax.ShapeDtypeStruct((e_pad,), jnp.float32),
        mesh=_sc_mesh(),
        scratch_types=[
            pltpu.VMEM_SHARED((NRP,), jnp.float32),
            pltpu.VMEM((CNT_SLICE,), jnp.float32),
            pltpu.VMEM((CNT_SLICE,), jnp.float32),
            pltpu.VMEM((CNT_SLICE,), jnp.float32),
            pltpu.VMEM((1, B), jnp.int32),
            pltpu.VMEM((B,), jnp.float32),
        ],
    )
    def k(cnt0_hbm, cnt1_hbm, cid_hbm, w_hbm,
          wbkt_sh, c0s, c1s, ws, cidbuf, wv):
        c = lax.axis_index("c")
        s = lax.axis_index("s")
        wid = s * NC + c
        base0 = wid * chunk

        sl = pl.ds(s * CNT_SLICE, CNT_SLICE)
        pltpu.sync_copy(cnt0_hbm.at[sl], c0s)
        pltpu.sync_copy(cnt1_hbm.at[sl], c1s)
        for i in range(CNT_SLICE // 16):
            s16 = pl.ds(16 * i, 16)
            ws[s16] = 1.0 / (c0s[s16] + c1s[s16])
        pltpu.sync_copy(ws, wbkt_sh.at[sl])
        plsc.subcore_barrier()

        def body(b, _):
            base = base0 + b * B
            pltpu.sync_copy(cid_hbm.at[pl.ds(base, B)], cidbuf.at[0])
            pltpu.sync_copy(wbkt_sh.at[cidbuf.at[0]], wv)
            pltpu.sync_copy(wv, w_hbm.at[pl.ds(base, B)])
            return ()

        lax.fori_loop(0, nbatch, body, ())

    return k


def _aggregate_sc(e_pad):
    chunk = e_pad // NW
    nbatch = chunk // B
    npairs = nbatch // 2
    tail = nbatch % 2

    @functools.partial(
        pl.kernel,
        out_type=(
            jax.ShapeDtypeStruct((AGG_ROWS, D), jnp.float32),
            jax.ShapeDtypeStruct((AGG_ROWS, D), jnp.float32),
        ),
        mesh=_sc_mesh(),
        scratch_types=[
            pltpu.VMEM_SHARED((AGG_ROWS, D), jnp.float32),
            pltpu.VMEM((1, B), jnp.int32),
            pltpu.VMEM((1, B), jnp.int32),
            pltpu.VMEM((1, B), jnp.int32),
            pltpu.VMEM((1, B), jnp.int32),
            pltpu.VMEM((B,), jnp.float32),
            pltpu.VMEM((B,), jnp.float32),
            pltpu.VMEM((B, D), jnp.float32),
            pltpu.VMEM((B, D), jnp.float32),
            pltpu.SemaphoreType.DMA,
            pltpu.SemaphoreType.DMA,
            pltpu.SemaphoreType.DMA,
            pltpu.SemaphoreType.DMA,
        ],
    )
    def k(gid_hbm, dst_hbm, w_hbm, h_hbm, zeros_hbm,
          agg0_hbm, agg1_hbm,
          agg_sh, gidx0, gidx1, dbuf0, dbuf1, wbuf0, wbuf1,
          rows0, rows1, semi0, semi1, semr0, semr1):
        c = lax.axis_index("c")
        s = lax.axis_index("s")
        wid = s * NC + c
        base0 = wid * chunk

        rsl = pl.ds(s * AGG_SLICE, AGG_SLICE)
        pltpu.sync_copy(zeros_hbm.at[rsl], agg_sh.at[rsl])
        plsc.subcore_barrier()

        slots = ((gidx0, dbuf0, wbuf0, rows0, semi0, semr0),
                 (gidx1, dbuf1, wbuf1, rows1, semi1, semr1))

        def prefetch(slot, base):
            gidx, dbuf, wbuf, rows, semi, semr = slot
            cps = (pltpu.async_copy(gid_hbm.at[pl.ds(base, B)],
                                    gidx.at[0], semi),
                   pltpu.async_copy(dst_hbm.at[pl.ds(base, B)],
                                    dbuf.at[0], semi),
                   pltpu.async_copy(w_hbm.at[pl.ds(base, B)], wbuf, semi))
            for cp in cps:
                cp.wait()
            pltpu.async_copy(h_hbm.at[gidx.at[0]], rows, semr)

        def scale_scatter(slot):
            gidx, dbuf, wbuf, rows, semi, semr = slot
            pltpu.make_async_copy(h_hbm.at[gidx.at[0]], rows, semr).wait()
            for g in range(B // 16):
                wv = wbuf[pl.ds(16 * g, 16)]
                for l in range(16):
                    w = wv[l]
                    e_row = 16 * g + l
                    for j in range(D // 16):
                        cs = pl.ds(16 * j, 16)
                        rows[e_row, cs] = rows[e_row, cs] * w
            pltpu.sync_copy(rows, agg_sh.at[dbuf.at[0]], add=True)

        prefetch(slots[0], base0)
        prefetch(slots[1], base0 + B)

        def body(p, _):
            for kk in range(2):
                scale_scatter(slots[kk])
                nxt = 2 * p + 2 + kk

                @pl.when(nxt < nbatch)
                def _():
                    prefetch(slots[kk], base0 + nxt * B)
            return ()

        lax.fori_loop(0, npairs, body, ())
        if tail:
            scale_scatter(slots[0])
        plsc.subcore_barrier()

        @pl.when(c == 0)
        def _():
            pltpu.sync_copy(agg_sh.at[rsl], agg0_hbm.at[rsl])

        @pl.when(c == 1)
        def _():
            pltpu.sync_copy(agg_sh.at[rsl], agg1_hbm.at[rsl])

    return k


def _w_combine(comp, basis2d):
    def body(c_ref, b_ref, o_ref):
        o_ref[...] = jnp.dot(c_ref[...], b_ref[...],
                             preferred_element_type=jnp.float32)

    return pl.pallas_call(
        body,
        out_shape=jax.ShapeDtypeStruct((R, D * D), jnp.float32),
    )(comp, basis2d)


_BN_H = 2000


def _h_all(x, w3d):
    def body(x_ref, w_ref, o_ref):
        o_ref[0] = jnp.dot(x_ref[...], w_ref[0],
                           preferred_element_type=jnp.float32)

    return pl.pallas_call(
        body,
        grid=(N // _BN_H, R),
        in_specs=[
            pl.BlockSpec((_BN_H, D), lambda n, r: (n, 0)),
            pl.BlockSpec((1, D, D), lambda n, r: (r, 0, 0)),
        ],
        out_specs=pl.BlockSpec((1, _BN_H, D), lambda n, r: (r, n, 0)),
        out_shape=jax.ShapeDtypeStruct((R, N, D), jnp.float32),
    )(x, w3d)


_BN_F = 2000


def _finish(x, agg0, agg1, root, bias, g, b):
    def body(x_ref, a0_ref, a1_ref, r_ref, bias_ref, g_ref, b_ref, o_ref):
        y = (a0_ref[...] + a1_ref[...]
             + jnp.dot(x_ref[...], r_ref[...],
                       preferred_element_type=jnp.float32)
             + bias_ref[0])
        m = jnp.mean(y, axis=-1, keepdims=True)
        yc = y - m
        v = jnp.mean(yc * yc, axis=-1, keepdims=True)
        o = yc / jnp.sqrt(v + 1e-5) * g_ref[0] + b_ref[0]
        o_ref[...] = jnp.maximum(o, 0.0)

    return pl.pallas_call(
        body,
        grid=(N // _BN_F,),
        in_specs=[
            pl.BlockSpec((_BN_F, D), lambda n: (n, 0)),
            pl.BlockSpec((_BN_F, D), lambda n: (n, 0)),
            pl.BlockSpec((_BN_F, D), lambda n: (n, 0)),
            pl.BlockSpec((D, D), lambda n: (0, 0)),
            pl.BlockSpec((1, D), lambda n: (0, 0)),
            pl.BlockSpec((1, D), lambda n: (0, 0)),
            pl.BlockSpec((1, D), lambda n: (0, 0)),
        ],
        out_specs=pl.BlockSpec((_BN_F, D), lambda n: (n, 0)),
        out_shape=jax.ShapeDtypeStruct((N, D), jnp.float32),
    )(x, agg0[:N], agg1[:N], root, bias.reshape(1, D), g.reshape(1, D),
      b.reshape(1, D))


def kernel(node_emb, edge_index, edge_type,
           comp0, basis0, root0, bias0, g0, b0,
           comp1, basis1, root1, bias1, g1, b1):
    e = edge_index.shape[1]
    e_pad = ((e + NW * B - 1) // (NW * B)) * (NW * B)
    pad = e_pad - e

    src = jnp.pad(edge_index[0].astype(jnp.int32), (0, pad))
    dst = jnp.pad(edge_index[1].astype(jnp.int32), (0, pad),
                  constant_values=N)
    typ = jnp.pad(edge_type.astype(jnp.int32), (0, pad))

    zeros_cnt = jnp.zeros((NRP,), jnp.float32)
    zeros_agg = jnp.zeros((AGG_ROWS, D), jnp.float32)

    cnt0, cnt1, gid, cid = _counts_sc(e_pad)(src, dst, typ, zeros_cnt)
    w = _weights_sc(e_pad)(cnt0, cnt1, cid)

    agg_fn = _aggregate_sc(e_pad)

    def layer(x, comp, basis, root, bias, g, b):
        w3d = _w_combine(comp, basis.reshape(R, D * D)).reshape(R, D, D)
        h = _h_all(x, w3d).reshape(R * N, D)
        a0, a1 = agg_fn(gid, dst, w, h, zeros_agg)
        return _finish(x, a0, a1, root, bias, g, b)

    x1 = layer(node_emb, comp0, basis0, root0, bias0, g0, b0)
    return layer(x1, comp1, basis1, root1, bias1, g1, b1)


# speedup vs baseline: 3.2323x; 1.1104x over previous
import functools

import jax
import jax.numpy as jnp
from jax import lax
from jax.experimental import pallas as pl
from jax.experimental.pallas import tpu as pltpu
from jax.experimental.pallas import tpu_sc as plsc

N = 10000
R = 24
D = 128

NC = 2
NS = 16
NW = NC * NS
B = 128

NR = N * R
NRP = 262144
CNT_SLICE = NRP // NS
AGG_ROWS = 10112
AGG_SLICE = AGG_ROWS // NS


def _sc_mesh():
    return plsc.VectorSubcoreMesh(core_axis_name="c", subcore_axis_name="s")


def _counts_sc(e_pad):
    chunk = e_pad // NW
    nbatch = chunk // B
    npairs = nbatch // 2
    tail = nbatch % 2

    @functools.partial(
        pl.kernel,
        out_type=(
            jax.ShapeDtypeStruct((NRP,), jnp.float32),
            jax.ShapeDtypeStruct((NRP,), jnp.float32),
            jax.ShapeDtypeStruct((e_pad,), jnp.int32),
            jax.ShapeDtypeStruct((e_pad,), jnp.int32),
        ),
        mesh=_sc_mesh(),
        scratch_types=[
            pltpu.VMEM_SHARED((NRP,), jnp.float32),
            pltpu.VMEM((B,), jnp.int32),
            pltpu.VMEM((B,), jnp.int32),
            pltpu.VMEM((B,), jnp.int32),
            pltpu.VMEM((B,), jnp.int32),
            pltpu.VMEM((B,), jnp.int32),
            pltpu.VMEM((B,), jnp.int32),
            pltpu.VMEM((1, B), jnp.int32),
            pltpu.VMEM((1, B), jnp.int32),
            pltpu.VMEM((1, B), jnp.int32),
            pltpu.VMEM((1, B), jnp.int32),
            pltpu.VMEM((B,), jnp.float32),
            pltpu.SemaphoreType.DMA,
            pltpu.SemaphoreType.DMA,
            pltpu.SemaphoreType.DMA,
            pltpu.SemaphoreType.DMA,
        ],
    )
    def k(src_hbm, dst_hbm, typ_hbm, zeros_hbm,
          cnt0_hbm, cnt1_hbm, gid_hbm, cid_hbm,
          cnt_sh, sbuf0, sbuf1, dbuf0, dbuf1, tbuf0, tbuf1,
          gidbuf0, gidbuf1, cidbuf0, cidbuf1, ones,
          seml0, seml1, sems0, sems1):
        c = lax.axis_index("c")
        s = lax.axis_index("s")
        wid = s * NC + c
        base0 = wid * chunk

        slots = ((sbuf0, dbuf0, tbuf0, gidbuf0, cidbuf0, seml0, sems0),
                 (sbuf1, dbuf1, tbuf1, gidbuf1, cidbuf1, seml1, sems1))

        pltpu.sync_copy(zeros_hbm.at[pl.ds(s * CNT_SLICE, CNT_SLICE)],
                        cnt_sh.at[pl.ds(s * CNT_SLICE, CNT_SLICE)])
        for g in range(B // 16):
            ones[pl.ds(16 * g, 16)] = jnp.ones((16,), jnp.float32)
        plsc.subcore_barrier()

        def load(slot, base):
            sbuf, dbuf, tbuf, gidbuf, cidbuf, seml, sems = slot
            pltpu.async_copy(src_hbm.at[pl.ds(base, B)], sbuf, seml)
            pltpu.async_copy(dst_hbm.at[pl.ds(base, B)], dbuf, seml)
            pltpu.async_copy(typ_hbm.at[pl.ds(base, B)], tbuf, seml)

        def process(slot, base, first):
            sbuf, dbuf, tbuf, gidbuf, cidbuf, seml, sems = slot
            pltpu.make_async_copy(src_hbm.at[pl.ds(base, B)], sbuf,
                                  seml).wait()
            pltpu.make_async_copy(dst_hbm.at[pl.ds(base, B)], dbuf,
                                  seml).wait()
            pltpu.make_async_copy(typ_hbm.at[pl.ds(base, B)], tbuf,
                                  seml).wait()
            if not first:
                pltpu.make_async_copy(gidbuf.at[0],
                                      gid_hbm.at[pl.ds(base, B)],
                                      sems).wait()
                pltpu.make_async_copy(cidbuf.at[0],
                                      cid_hbm.at[pl.ds(base, B)],
                                      sems).wait()
            for g in range(B // 16):
                sl16 = pl.ds(16 * g, 16)
                sv = sbuf[sl16]
                dv = dbuf[sl16]
                tv = tbuf[sl16]
                gidbuf[0, sl16] = tv * N + sv
                cidbuf[0, sl16] = dv * R + tv
            pltpu.sync_copy(ones, cnt_sh.at[cidbuf.at[0]], add=True)
            pltpu.async_copy(gidbuf.at[0], gid_hbm.at[pl.ds(base, B)], sems)
            pltpu.async_copy(cidbuf.at[0], cid_hbm.at[pl.ds(base, B)], sems)

        load(slots[0], base0)
        load(slots[1], base0 + B)

        def body(p, _):
            for kk in range(2):
                b = 2 * p + kk
                base = base0 + b * B

                @pl.when(p == 0)
                def _():
                    process(slots[kk], base, True)

                @pl.when(p > 0)
                def _():
                    process(slots[kk], base, False)

                nxt = b + 2

                @pl.when(nxt < nbatch)
                def _():
                    load(slots[kk], base0 + nxt * B)
            return ()

        lax.fori_loop(0, npairs, body, ())
        if tail:
            process(slots[0], base0 + (nbatch - 1) * B, npairs == 0)
        for kk in range(2):
            sbuf, dbuf, tbuf, gidbuf, cidbuf, seml, sems = slots[kk]
            pltpu.make_async_copy(gidbuf.at[0], gid_hbm.at[pl.ds(base0, B)],
                                  sems).wait()
            pltpu.make_async_copy(cidbuf.at[0], cid_hbm.at[pl.ds(base0, B)],
                                  sems).wait()
        plsc.subcore_barrier()

        sl = pl.ds(s * CNT_SLICE, CNT_SLICE)

        @pl.when(c == 0)
        def _():
            pltpu.sync_copy(cnt_sh.at[sl], cnt0_hbm.at[sl])

        @pl.when(c == 1)
        def _():
            pltpu.sync_copy(cnt_sh.at[sl], cnt1_hbm.at[sl])

    return k


def _weights_sc(e_pad):
    chunk = e_pad // NW
    nbatch = chunk // B

    @functools.partial(
        pl.kernel,
        out_type=jax.ShapeDtypeStruct((e_pad,), jnp.float32),
        mesh=_sc_mesh(),
        scratch_types=[
            pltpu.VMEM_SHARED((NRP,), jnp.float32),
            pltpu.VMEM((CNT_SLICE,), jnp.float32),
            pltpu.VMEM((CNT_SLICE,), jnp.float32),
            pltpu.VMEM((CNT_SLICE,), jnp.float32),
            pltpu.VMEM((1, B), jnp.int32),
            pltpu.VMEM((B,), jnp.float32),
        ],
    )
    def k(cnt0_hbm, cnt1_hbm, cid_hbm, w_hbm,
          wbkt_sh, c0s, c1s, ws, cidbuf, wv):
        c = lax.axis_index("c")
        s = lax.axis_index("s")
        wid = s * NC + c
        base0 = wid * chunk

        sl = pl.ds(s * CNT_SLICE, CNT_SLICE)
        pltpu.sync_copy(cnt0_hbm.at[sl], c0s)
        pltpu.sync_copy(cnt1_hbm.at[sl], c1s)
        for i in range(CNT_SLICE // 16):
            s16 = pl.ds(16 * i, 16)
            ws[s16] = 1.0 / (c0s[s16] + c1s[s16])
        pltpu.sync_copy(ws, wbkt_sh.at[sl])
        plsc.subcore_barrier()

        def body(b, _):
            base = base0 + b * B
            pltpu.sync_copy(cid_hbm.at[pl.ds(base, B)], cidbuf.at[0])
            pltpu.sync_copy(wbkt_sh.at[cidbuf.at[0]], wv)
            pltpu.sync_copy(wv, w_hbm.at[pl.ds(base, B)])
            return ()

        lax.fori_loop(0, nbatch, body, ())

    return k


def _aggregate_sc(e_pad):
    chunk = e_pad // NW
    nbatch = chunk // B
    npairs = nbatch // 2
    tail = nbatch % 2

    @functools.partial(
        pl.kernel,
        out_type=(
            jax.ShapeDtypeStruct((AGG_ROWS, D), jnp.float32),
            jax.ShapeDtypeStruct((AGG_ROWS, D), jnp.float32),
        ),
        mesh=_sc_mesh(),
        scratch_types=[
            pltpu.VMEM_SHARED((AGG_ROWS, D), jnp.float32),
            pltpu.VMEM((1, B), jnp.int32),
            pltpu.VMEM((1, B), jnp.int32),
            pltpu.VMEM((1, B), jnp.int32),
            pltpu.VMEM((1, B), jnp.int32),
            pltpu.VMEM((B,), jnp.float32),
            pltpu.VMEM((B,), jnp.float32),
            pltpu.VMEM((B, D), jnp.float32),
            pltpu.VMEM((B, D), jnp.float32),
            pltpu.SemaphoreType.DMA,
            pltpu.SemaphoreType.DMA,
            pltpu.SemaphoreType.DMA,
            pltpu.SemaphoreType.DMA,
        ],
    )
    def k(gid_hbm, dst_hbm, w_hbm, h_hbm, zeros_hbm,
          agg0_hbm, agg1_hbm,
          agg_sh, gidx0, gidx1, dbuf0, dbuf1, wbuf0, wbuf1,
          rows0, rows1, semi0, semi1, semr0, semr1):
        c = lax.axis_index("c")
        s = lax.axis_index("s")
        wid = s * NC + c
        base0 = wid * chunk

        rsl = pl.ds(s * AGG_SLICE, AGG_SLICE)
        pltpu.sync_copy(zeros_hbm.at[rsl], agg_sh.at[rsl])
        plsc.subcore_barrier()

        slots = ((gidx0, dbuf0, wbuf0, rows0, semi0, semr0),
                 (gidx1, dbuf1, wbuf1, rows1, semi1, semr1))

        def prefetch(slot, base):
            gidx, dbuf, wbuf, rows, semi, semr = slot
            cps = (pltpu.async_copy(gid_hbm.at[pl.ds(base, B)],
                                    gidx.at[0], semi),
                   pltpu.async_copy(dst_hbm.at[pl.ds(base, B)],
                                    dbuf.at[0], semi),
                   pltpu.async_copy(w_hbm.at[pl.ds(base, B)], wbuf, semi))
            for cp in cps:
                cp.wait()
            pltpu.async_copy(h_hbm.at[gidx.at[0]], rows, semr)

        def scale_scatter(slot):
            gidx, dbuf, wbuf, rows, semi, semr = slot
            pltpu.make_async_copy(h_hbm.at[gidx.at[0]], rows, semr).wait()
            for g in range(B // 16):
                wv = wbuf[pl.ds(16 * g, 16)]
                for l in range(16):
                    w = wv[l]
                    e_row = 16 * g + l
                    for j in range(D // 16):
                        cs = pl.ds(16 * j, 16)
                        rows[e_row, cs] = rows[e_row, cs] * w
            pltpu.sync_copy(rows, agg_sh.at[dbuf.at[0]], add=True)

        prefetch(slots[0], base0)
        prefetch(slots[1], base0 + B)

        def body(p, _):
            for kk in range(2):
                scale_scatter(slots[kk])
                nxt = 2 * p + 2 + kk

                @pl.when(nxt < nbatch)
                def _():
                    prefetch(slots[kk], base0 + nxt * B)
            return ()

        lax.fori_loop(0, npairs, body, ())
        if tail:
            scale_scatter(slots[0])
        plsc.subcore_barrier()

        @pl.when(c == 0)
        def _():
            pltpu.sync_copy(agg_sh.at[rsl], agg0_hbm.at[rsl])

        @pl.when(c == 1)
        def _():
            pltpu.sync_copy(agg_sh.at[rsl], agg1_hbm.at[rsl])

    return k


def _w_combine(comp, basis2d):
    def body(c_ref, b_ref, o_ref):
        o_ref[...] = jnp.dot(c_ref[...], b_ref[...],
                             preferred_element_type=jnp.float32)

    return pl.pallas_call(
        body,
        out_shape=jax.ShapeDtypeStruct((R, D * D), jnp.float32),
    )(comp, basis2d)


_BN_H = 2000


def _h_all(x, w3d):
    def body(x_ref, w_ref, o_ref):
        o_ref[0] = jnp.dot(x_ref[...], w_ref[0],
                           preferred_element_type=jnp.float32)

    return pl.pallas_call(
        body,
        grid=(N // _BN_H, R),
        in_specs=[
            pl.BlockSpec((_BN_H, D), lambda n, r: (n, 0)),
            pl.BlockSpec((1, D, D), lambda n, r: (r, 0, 0)),
        ],
        out_specs=pl.BlockSpec((1, _BN_H, D), lambda n, r: (r, n, 0)),
        out_shape=jax.ShapeDtypeStruct((R, N, D), jnp.float32),
    )(x, w3d)


_BN_F = 2000


def _finish(x, agg0, agg1, root, bias, g, b):
    def body(x_ref, a0_ref, a1_ref, r_ref, bias_ref, g_ref, b_ref, o_ref):
        y = (a0_ref[...] + a1_ref[...]
             + jnp.dot(x_ref[...], r_ref[...],
                       preferred_element_type=jnp.float32)
             + bias_ref[0])
        m = jnp.mean(y, axis=-1, keepdims=True)
        yc = y - m
        v = jnp.mean(yc * yc, axis=-1, keepdims=True)
        o = yc / jnp.sqrt(v + 1e-5) * g_ref[0] + b_ref[0]
        o_ref[...] = jnp.maximum(o, 0.0)

    return pl.pallas_call(
        body,
        grid=(N // _BN_F,),
        in_specs=[
            pl.BlockSpec((_BN_F, D), lambda n: (n, 0)),
            pl.BlockSpec((_BN_F, D), lambda n: (n, 0)),
            pl.BlockSpec((_BN_F, D), lambda n: (n, 0)),
            pl.BlockSpec((D, D), lambda n: (0, 0)),
            pl.BlockSpec((1, D), lambda n: (0, 0)),
            pl.BlockSpec((1, D), lambda n: (0, 0)),
            pl.BlockSpec((1, D), lambda n: (0, 0)),
        ],
        out_specs=pl.BlockSpec((_BN_F, D), lambda n: (n, 0)),
        out_shape=jax.ShapeDtypeStruct((N, D), jnp.float32),
    )(x, agg0[:N], agg1[:N], root, bias.reshape(1, D), g.reshape(1, D),
      b.reshape(1, D))


def kernel(node_emb, edge_index, edge_type,
           comp0, basis0, root0, bias0, g0, b0,
           comp1, basis1, root1, bias1, g1, b1):
    e = edge_index.shape[1]
    e_pad = ((e + NW * B - 1) // (NW * B)) * (NW * B)
    pad = e_pad - e

    src = jnp.pad(edge_index[0].astype(jnp.int32), (0, pad))
    dst = jnp.pad(edge_index[1].astype(jnp.int32), (0, pad),
                  constant_values=N)
    typ = jnp.pad(edge_type.astype(jnp.int32), (0, pad))

    zeros_cnt = jnp.zeros((NRP,), jnp.float32)
    zeros_agg = jnp.zeros((AGG_ROWS, D), jnp.float32)

    cnt0, cnt1, gid, cid = _counts_sc(e_pad)(src, dst, typ, zeros_cnt)
    w = _weights_sc(e_pad)(cnt0, cnt1, cid)

    agg_fn = _aggregate_sc(e_pad)

    def layer(x, comp, basis, root, bias, g, b):
        w3d = _w_combine(comp, basis.reshape(R, D * D)).reshape(R, D, D)
        h = _h_all(x, w3d).reshape(R * N, D)
        a0, a1 = agg_fn(gid, dst, w, h, zeros_agg)
        return _finish(x, a0, a1, root, bias, g, b)

    x1 = layer(node_emb, comp0, basis0, root0, bias0, g0, b0)
    return layer(x1, comp1, basis1, root1, bias1, g1, b1)


# speedup vs baseline: 3.4287x; 1.0608x over previous
import functools

import jax
import jax.numpy as jnp
from jax import lax
from jax.experimental import pallas as pl
from jax.experimental.pallas import tpu as pltpu
from jax.experimental.pallas import tpu_sc as plsc

N = 10000
R = 24
D = 128

NC = 2
NS = 16
NW = NC * NS
B = 128

NR = N * R
NRP = 262144
CNT_SLICE = NRP // NS
AGG_ROWS = 10112
AGG_SLICE = AGG_ROWS // NS


def _sc_mesh():
    return plsc.VectorSubcoreMesh(core_axis_name="c", subcore_axis_name="s")


def _counts_sc(e_pad):
    chunk = e_pad // NW
    nbatch = chunk // B
    npairs = nbatch // 2
    tail = nbatch % 2

    @functools.partial(
        pl.kernel,
        out_type=(
            jax.ShapeDtypeStruct((NRP,), jnp.float32),
            jax.ShapeDtypeStruct((NRP,), jnp.float32),
            jax.ShapeDtypeStruct((e_pad,), jnp.int32),
            jax.ShapeDtypeStruct((e_pad,), jnp.int32),
        ),
        mesh=_sc_mesh(),
        scratch_types=[
            pltpu.VMEM_SHARED((NRP,), jnp.float32),
            pltpu.VMEM((B,), jnp.int32),
            pltpu.VMEM((B,), jnp.int32),
            pltpu.VMEM((B,), jnp.int32),
            pltpu.VMEM((B,), jnp.int32),
            pltpu.VMEM((B,), jnp.int32),
            pltpu.VMEM((B,), jnp.int32),
            pltpu.VMEM((1, B), jnp.int32),
            pltpu.VMEM((1, B), jnp.int32),
            pltpu.VMEM((1, B), jnp.int32),
            pltpu.VMEM((1, B), jnp.int32),
            pltpu.VMEM((B,), jnp.float32),
            pltpu.SemaphoreType.DMA,
            pltpu.SemaphoreType.DMA,
            pltpu.SemaphoreType.DMA,
            pltpu.SemaphoreType.DMA,
        ],
    )
    def k(src_hbm, dst_hbm, typ_hbm, zeros_hbm,
          cnt0_hbm, cnt1_hbm, gid_hbm, cid_hbm,
          cnt_sh, sbuf0, sbuf1, dbuf0, dbuf1, tbuf0, tbuf1,
          gidbuf0, gidbuf1, cidbuf0, cidbuf1, ones,
          seml0, seml1, sems0, sems1):
        c = lax.axis_index("c")
        s = lax.axis_index("s")
        wid = s * NC + c
        base0 = wid * chunk

        slots = ((sbuf0, dbuf0, tbuf0, gidbuf0, cidbuf0, seml0, sems0),
                 (sbuf1, dbuf1, tbuf1, gidbuf1, cidbuf1, seml1, sems1))

        pltpu.sync_copy(zeros_hbm.at[pl.ds(s * CNT_SLICE, CNT_SLICE)],
                        cnt_sh.at[pl.ds(s * CNT_SLICE, CNT_SLICE)])
        for g in range(B // 16):
            ones[pl.ds(16 * g, 16)] = jnp.ones((16,), jnp.float32)
        plsc.subcore_barrier()

        def load(slot, base):
            sbuf, dbuf, tbuf, gidbuf, cidbuf, seml, sems = slot
            pltpu.async_copy(src_hbm.at[pl.ds(base, B)], sbuf, seml)
            pltpu.async_copy(dst_hbm.at[pl.ds(base, B)], dbuf, seml)
            pltpu.async_copy(typ_hbm.at[pl.ds(base, B)], tbuf, seml)

        def process(slot, base, first):
            sbuf, dbuf, tbuf, gidbuf, cidbuf, seml, sems = slot
            pltpu.make_async_copy(src_hbm.at[pl.ds(base, B)], sbuf,
                                  seml).wait()
            pltpu.make_async_copy(dst_hbm.at[pl.ds(base, B)], dbuf,
                                  seml).wait()
            pltpu.make_async_copy(typ_hbm.at[pl.ds(base, B)], tbuf,
                                  seml).wait()
            if not first:
                pltpu.make_async_copy(gidbuf.at[0],
                                      gid_hbm.at[pl.ds(base, B)],
                                      sems).wait()
                pltpu.make_async_copy(cidbuf.at[0],
                                      cid_hbm.at[pl.ds(base, B)],
                                      sems).wait()
            for g in range(B // 16):
                sl16 = pl.ds(16 * g, 16)
                sv = sbuf[sl16]
                dv = dbuf[sl16]
                tv = tbuf[sl16]
                gidbuf[0, sl16] = tv * N + sv
                cidbuf[0, sl16] = dv * R + tv
            pltpu.sync_copy(ones, cnt_sh.at[cidbuf.at[0]], add=True)
            pltpu.async_copy(gidbuf.at[0], gid_hbm.at[pl.ds(base, B)], sems)
            pltpu.async_copy(cidbuf.at[0], cid_hbm.at[pl.ds(base, B)], sems)

        load(slots[0], base0)
        load(slots[1], base0 + B)

        def body(p, _):
            for kk in range(2):
                b = 2 * p + kk
                base = base0 + b * B

                @pl.when(p == 0)
                def _():
                    process(slots[kk], base, True)

                @pl.when(p > 0)
                def _():
                    process(slots[kk], base, False)

                nxt = b + 2

                @pl.when(nxt < nbatch)
                def _():
                    load(slots[kk], base0 + nxt * B)
            return ()

        lax.fori_loop(0, npairs, body, ())
        if tail:
            process(slots[0], base0 + (nbatch - 1) * B, npairs == 0)
        for kk in range(2):
            sbuf, dbuf, tbuf, gidbuf, cidbuf, seml, sems = slots[kk]
            pltpu.make_async_copy(gidbuf.at[0], gid_hbm.at[pl.ds(base0, B)],
                                  sems).wait()
            pltpu.make_async_copy(cidbuf.at[0], cid_hbm.at[pl.ds(base0, B)],
                                  sems).wait()
        plsc.subcore_barrier()

        sl = pl.ds(s * CNT_SLICE, CNT_SLICE)

        @pl.when(c == 0)
        def _():
            pltpu.sync_copy(cnt_sh.at[sl], cnt0_hbm.at[sl])

        @pl.when(c == 1)
        def _():
            pltpu.sync_copy(cnt_sh.at[sl], cnt1_hbm.at[sl])

    return k


def _weights_sc(e_pad):
    chunk = e_pad // NW
    nbatch = chunk // B

    @functools.partial(
        pl.kernel,
        out_type=jax.ShapeDtypeStruct((e_pad,), jnp.float32),
        mesh=_sc_mesh(),
        scratch_types=[
            pltpu.VMEM_SHARED((NRP,), jnp.float32),
            pltpu.VMEM((CNT_SLICE,), jnp.float32),
            pltpu.VMEM((CNT_SLICE,), jnp.float32),
            pltpu.VMEM((CNT_SLICE,), jnp.float32),
            pltpu.VMEM((1, B), jnp.int32),
            pltpu.VMEM((B,), jnp.float32),
        ],
    )
    def k(cnt0_hbm, cnt1_hbm, cid_hbm, w_hbm,
          wbkt_sh, c0s, c1s, ws, cidbuf, wv):
        c = lax.axis_index("c")
        s = lax.axis_index("s")
        wid = s * NC + c
        base0 = wid * chunk

        sl = pl.ds(s * CNT_SLICE, CNT_SLICE)
        pltpu.sync_copy(cnt0_hbm.at[sl], c0s)
        pltpu.sync_copy(cnt1_hbm.at[sl], c1s)
        for i in range(CNT_SLICE // 16):
            s16 = pl.ds(16 * i, 16)
            ws[s16] = 1.0 / (c0s[s16] + c1s[s16])
        pltpu.sync_copy(ws, wbkt_sh.at[sl])
        plsc.subcore_barrier()

        def body(b, _):
            base = base0 + b * B
            pltpu.sync_copy(cid_hbm.at[pl.ds(base, B)], cidbuf.at[0])
            pltpu.sync_copy(wbkt_sh.at[cidbuf.at[0]], wv)
            pltpu.sync_copy(wv, w_hbm.at[pl.ds(base, B)])
            return ()

        lax.fori_loop(0, nbatch, body, ())

    return k


def _aggregate_sc(e_pad):
    BA = 64
    chunk = e_pad // NW
    nbatch = chunk // BA
    npairs = nbatch // 2
    tail = nbatch % 2

    @functools.partial(
        pl.kernel,
        out_type=(
            jax.ShapeDtypeStruct((AGG_ROWS, D), jnp.float32),
            jax.ShapeDtypeStruct((AGG_ROWS, D), jnp.float32),
        ),
        mesh=_sc_mesh(),
        scratch_types=[
            pltpu.VMEM_SHARED((AGG_ROWS, D), jnp.float32),
            pltpu.VMEM((1, BA), jnp.int32),
            pltpu.VMEM((1, BA), jnp.int32),
            pltpu.VMEM((1, BA), jnp.int32),
            pltpu.VMEM((1, BA), jnp.int32),
            pltpu.VMEM((1, BA), jnp.int32),
            pltpu.VMEM((1, BA), jnp.int32),
            pltpu.VMEM((BA,), jnp.float32),
            pltpu.VMEM((BA,), jnp.float32),
            pltpu.VMEM((BA, D), jnp.float32),
            pltpu.VMEM((BA, D), jnp.float32),
            pltpu.VMEM((BA, D), jnp.float32),
            pltpu.VMEM((BA, D), jnp.float32),
            pltpu.SemaphoreType.DMA,
            pltpu.SemaphoreType.DMA,
            pltpu.SemaphoreType.DMA,
            pltpu.SemaphoreType.DMA,
            pltpu.SemaphoreType.DMA,
            pltpu.SemaphoreType.DMA,
        ],
    )
    def k(gid_hbm, dst_hbm, w_hbm, h_hbm, zeros_hbm,
          agg0_hbm, agg1_hbm,
          agg_sh, gidx0, gidx1, dbuf0, dbuf1, dscat0, dscat1, wbuf0, wbuf1,
          rows0, rows1, rowsf0, rowsf1,
          semi0, semi1, semr0, semr1, sems0, sems1):
        c = lax.axis_index("c")
        s = lax.axis_index("s")
        wid = s * NC + c
        base0 = wid * chunk

        rsl = pl.ds(s * AGG_SLICE, AGG_SLICE)
        pltpu.sync_copy(zeros_hbm.at[rsl], agg_sh.at[rsl])
        plsc.subcore_barrier()

        slots = ((gidx0, dbuf0, dscat0, wbuf0, rows0, rowsf0,
                  semi0, semr0, sems0),
                 (gidx1, dbuf1, dscat1, wbuf1, rows1, rowsf1,
                  semi1, semr1, sems1))

        def prefetch(slot, base):
            gidx, dbuf, dscat, wbuf, rows, rowsf, semi, semr, sems = slot
            cps = (pltpu.async_copy(gid_hbm.at[pl.ds(base, BA)],
                                    gidx.at[0], semi),
                   pltpu.async_copy(dst_hbm.at[pl.ds(base, BA)],
                                    dbuf.at[0], semi),
                   pltpu.async_copy(w_hbm.at[pl.ds(base, BA)], wbuf, semi))
            for cp in cps:
                cp.wait()
            pltpu.async_copy(h_hbm.at[gidx.at[0]], rows, semr)

        def scale_scatter(slot, first):
            gidx, dbuf, dscat, wbuf, rows, rowsf, semi, semr, sems = slot
            if not first:
                pltpu.make_async_copy(rowsf, agg_sh.at[dscat.at[0]],
                                      sems).wait()
            pltpu.make_async_copy(h_hbm.at[gidx.at[0]], rows, semr).wait()
            for g in range(BA // 16):
                sl16 = pl.ds(16 * g, 16)
                dscat[0, sl16] = dbuf[0, sl16]
                wv = wbuf[sl16]
                for l in range(16):
                    w = wv[l]
                    e_row = 16 * g + l
                    for j in range(D // 16):
                        cs = pl.ds(16 * j, 16)
                        rowsf[e_row, cs] = rows[e_row, cs] * w
            pltpu.async_copy(rowsf, agg_sh.at[dscat.at[0]], sems, add=True)

        prefetch(slots[0], base0)
        prefetch(slots[1], base0 + BA)

        def body(p, _):
            for kk in range(2):

                @pl.when(p == 0)
                def _():
                    scale_scatter(slots[kk], True)

                @pl.when(p > 0)
                def _():
                    scale_scatter(slots[kk], False)

                nxt = 2 * p + 2 + kk

                @pl.when(nxt < nbatch)
                def _():
                    prefetch(slots[kk], base0 + nxt * BA)
            return ()

        lax.fori_loop(0, npairs, body, ())
        if tail:
            scale_scatter(slots[0], npairs == 0)
        for kk in range(2):
            gidx, dbuf, dscat, wbuf, rows, rowsf, semi, semr, sems = slots[kk]
            pltpu.make_async_copy(rowsf, agg_sh.at[dscat.at[0]], sems).wait()
        plsc.subcore_barrier()

        @pl.when(c == 0)
        def _():
            pltpu.sync_copy(agg_sh.at[rsl], agg0_hbm.at[rsl])

        @pl.when(c == 1)
        def _():
            pltpu.sync_copy(agg_sh.at[rsl], agg1_hbm.at[rsl])

    return k


def _w_combine(comp, basis2d):
    def body(c_ref, b_ref, o_ref):
        o_ref[...] = jnp.dot(c_ref[...], b_ref[...],
                             preferred_element_type=jnp.float32)

    return pl.pallas_call(
        body,
        out_shape=jax.ShapeDtypeStruct((R, D * D), jnp.float32),
    )(comp, basis2d)


_BN_H = 2000


def _h_all(x, w3d):
    def body(x_ref, w_ref, o_ref):
        o_ref[0] = jnp.dot(x_ref[...], w_ref[0],
                           preferred_element_type=jnp.float32)

    return pl.pallas_call(
        body,
        grid=(N // _BN_H, R),
        in_specs=[
            pl.BlockSpec((_BN_H, D), lambda n, r: (n, 0)),
            pl.BlockSpec((1, D, D), lambda n, r: (r, 0, 0)),
        ],
        out_specs=pl.BlockSpec((1, _BN_H, D), lambda n, r: (r, n, 0)),
        out_shape=jax.ShapeDtypeStruct((R, N, D), jnp.float32),
    )(x, w3d)


_BN_F = 2000


def _finish(x, agg0, agg1, root, bias, g, b):
    def body(x_ref, a0_ref, a1_ref, r_ref, bias_ref, g_ref, b_ref, o_ref):
        y = (a0_ref[...] + a1_ref[...]
             + jnp.dot(x_ref[...], r_ref[...],
                       preferred_element_type=jnp.float32)
             + bias_ref[0])
        m = jnp.mean(y, axis=-1, keepdims=True)
        yc = y - m
        v = jnp.mean(yc * yc, axis=-1, keepdims=True)
        o = yc / jnp.sqrt(v + 1e-5) * g_ref[0] + b_ref[0]
        o_ref[...] = jnp.maximum(o, 0.0)

    return pl.pallas_call(
        body,
        grid=(N // _BN_F,),
        in_specs=[
            pl.BlockSpec((_BN_F, D), lambda n: (n, 0)),
            pl.BlockSpec((_BN_F, D), lambda n: (n, 0)),
            pl.BlockSpec((_BN_F, D), lambda n: (n, 0)),
            pl.BlockSpec((D, D), lambda n: (0, 0)),
            pl.BlockSpec((1, D), lambda n: (0, 0)),
            pl.BlockSpec((1, D), lambda n: (0, 0)),
            pl.BlockSpec((1, D), lambda n: (0, 0)),
        ],
        out_specs=pl.BlockSpec((_BN_F, D), lambda n: (n, 0)),
        out_shape=jax.ShapeDtypeStruct((N, D), jnp.float32),
    )(x, agg0[:N], agg1[:N], root, bias.reshape(1, D), g.reshape(1, D),
      b.reshape(1, D))


def kernel(node_emb, edge_index, edge_type,
           comp0, basis0, root0, bias0, g0, b0,
           comp1, basis1, root1, bias1, g1, b1):
    e = edge_index.shape[1]
    e_pad = ((e + NW * B - 1) // (NW * B)) * (NW * B)
    pad = e_pad - e

    src = jnp.pad(edge_index[0].astype(jnp.int32), (0, pad))
    dst = jnp.pad(edge_index[1].astype(jnp.int32), (0, pad),
                  constant_values=N)
    typ = jnp.pad(edge_type.astype(jnp.int32), (0, pad))

    zeros_cnt = jnp.zeros((NRP,), jnp.float32)
    zeros_agg = jnp.zeros((AGG_ROWS, D), jnp.float32)

    cnt0, cnt1, gid, cid = _counts_sc(e_pad)(src, dst, typ, zeros_cnt)
    w = _weights_sc(e_pad)(cnt0, cnt1, cid)

    agg_fn = _aggregate_sc(e_pad)

    def layer(x, comp, basis, root, bias, g, b):
        w3d = _w_combine(comp, basis.reshape(R, D * D)).reshape(R, D, D)
        h = _h_all(x, w3d).reshape(R * N, D)
        a0, a1 = agg_fn(gid, dst, w, h, zeros_agg)
        return _finish(x, a0, a1, root, bias, g, b)

    x1 = layer(node_emb, comp0, basis0, root0, bias0, g0, b0)
    return layer(x1, comp1, basis1, root1, bias1, g1, b1)
